# baseline (speedup 1.0000x reference)
import functools

import jax
import jax.numpy as jnp
from jax import lax
from jax.experimental import pallas as pl
from jax.experimental.pallas import tpu as pltpu
from jax.experimental.pallas import tpu_sc as plsc

VOCAB = 1000
DIM = 64
BATCH = 16384
SEQ = 200

VPAD = 1024
NC, NS, L = 2, 16, 16
NW = NC * NS
NH = 4
HBATCH = BATCH // NH
ROWS_PER_W = HBATCH // NW
CHUNK = 32
NCHUNK = ROWS_PER_W // CHUNK
FULL_VECS = SEQ // L
TAIL = SEQ - FULL_VECS * L
NCB = VPAD // 128
BT = 256
NBLK_H = HBATCH // BT
CPB = BT // CHUNK

_mesh = plsc.VectorSubcoreMesh(core_axis_name="c", subcore_axis_name="s")


def _make_sc_counts(base_row):
    @functools.partial(
        pl.kernel,
        mesh=_mesh,
        out_type=jax.ShapeDtypeStruct((NBLK_H, NCB, CPB, CHUNK, 128), jnp.float32),
        scratch_types=[
            pltpu.VMEM((CHUNK, SEQ), jnp.int32),
            pltpu.VMEM((CHUNK, SEQ), jnp.int32),
            pltpu.VMEM((NCB, CHUNK, 128), jnp.float32),
            pltpu.VMEM((NCB, CHUNK, 128), jnp.float32),
            pltpu.SemaphoreType.DMA,
            pltpu.SemaphoreType.DMA,
        ],
        compiler_params=pltpu.CompilerParams(needs_layout_passes=False),
    )
    def _sc_counts(tok_hbm, counts_hbm, tok_a, tok_b, cnt_a, cnt_b, sem_a, sem_b):
        wid = lax.axis_index("s") * NC + lax.axis_index("c")
        row0 = wid * ROWS_PER_W

        zerosf = jnp.zeros((L,), jnp.float32)
        iota16 = lax.iota(jnp.int32, L)
        tail_mask = iota16 >= (L - TAIL)
        plus1 = jnp.full((L,), 1.0, jnp.float32)
        minus1 = jnp.full((L,), -1.0, jnp.float32)

        def zero_buf(cnt_v):
            @plsc.parallel_loop(0, NCB * CHUNK, unroll=2)
            def _(i):
                cb = i // CHUNK
                r = i % CHUNK
                for j in range(128 // L):
                    cnt_v[cb, r, pl.ds(j * L, L)] = zerosf

        def scatter_pass(tok_v, cnt_v, val_vec):
            @plsc.parallel_loop(0, CHUNK, unroll=2)
            def _(r):
                rows = jnp.full((L,), r, jnp.int32)
                for j in range(FULL_VECS + 1):
                    if j < FULL_VECS:
                        tok = tok_v[r, pl.ds(j * L, L)]
                        mask = None
                    else:
                        tok = tok_v[r, pl.ds(SEQ - L, L)]
                        mask = tail_mask
                    cbv = lax.shift_right_logical(tok, 7)
                    lo = tok & jnp.int32(0x7F)
                    if mask is None:
                        plsc.addupdate_scatter(cnt_v, [cbv, rows, lo], val_vec)
                    else:
                        plsc.addupdate_scatter(
                            cnt_v, [cbv, rows, lo], val_vec, mask=mask
                        )

        def load_tokens(c, tok_v):
            start = base_row + row0 + c * CHUNK
            pltpu.sync_copy(tok_hbm.at[pl.ds(start, CHUNK), :], tok_v)

        def cnt_dst(c):
            crow = row0 + c * CHUNK
            return counts_hbm.at[crow // BT, :, (crow % BT) // CHUNK]

        def start_cnt_dma(c, cnt_v, sem):
            pltpu.make_async_copy(cnt_v, cnt_dst(c), sem).start()

        def wait_cnt_dma(cnt_v, sem):
            pltpu.make_async_copy(cnt_v, cnt_dst(0), sem).wait()

        zero_buf(cnt_a)
        zero_buf(cnt_b)
        load_tokens(0, tok_a)
        scatter_pass(tok_a, cnt_a, plus1)
        start_cnt_dma(0, cnt_a, sem_a)
        load_tokens(1, tok_b)
        scatter_pass(tok_b, cnt_b, plus1)
        start_cnt_dma(1, cnt_b, sem_b)

        def chunk_pair(g, _):
            for c, tok_v, cnt_v, sem in (
                (2 * g, tok_a, cnt_a, sem_a),
                (2 * g + 1, tok_b, cnt_b, sem_b),
            ):
                wait_cnt_dma(cnt_v, sem)
                scatter_pass(tok_v, cnt_v, minus1)
                load_tokens(c, tok_v)
                scatter_pass(tok_v, cnt_v, plus1)
                start_cnt_dma(c, cnt_v, sem)
            return _

        lax.fori_loop(1, NCHUNK // 2, chunk_pair, None)
        wait_cnt_dma(cnt_a, sem_a)
        wait_cnt_dma(cnt_b, sem_b)

    return _sc_counts


_sc_counts_halves = tuple(_make_sc_counts(h * HBATCH) for h in range(NH))


def _tc_matmul_body(counts_ref, emb_ref, out_ref):
    x = counts_ref[...]
    emb = emb_ref[...]
    acc = jnp.dot(x[0:BT, :], emb[0:128, :], preferred_element_type=jnp.float32)
    for cb in range(1, NCB):
        acc += jnp.dot(
            x[cb * BT : (cb + 1) * BT, :],
            emb[cb * 128 : (cb + 1) * 128, :],
            preferred_element_type=jnp.float32,
        )
    out_ref[...] = acc * (1.0 / SEQ)


def _tc_matmul(counts2d, emb_pad):
    return pl.pallas_call(
        _tc_matmul_body,
        grid=(NBLK_H,),
        in_specs=[
            pl.BlockSpec((NCB * BT, 128), lambda i: (i, 0)),
            pl.BlockSpec((VPAD, DIM), lambda i: (0, 0)),
        ],
        out_specs=pl.BlockSpec((BT, DIM), lambda i: (i, 0)),
        out_shape=jax.ShapeDtypeStruct((HBATCH, DIM), jnp.float32),
    )(counts2d, emb_pad)


def kernel(token_ids, emb_weight):
    emb_pad = jnp.pad(emb_weight, ((0, VPAD - VOCAB), (0, 0)))
    outs = []
    for h in range(NH):
        counts = _sc_counts_halves[h](token_ids)
        counts2d = counts.reshape(NBLK_H * NCB * BT, 128)
        outs.append(_tc_matmul(counts2d, emb_pad))
    return jnp.concatenate(outs, axis=0)


# speedup vs baseline: 1.1407x; 1.1407x over previous
import functools

import jax
import jax.numpy as jnp
from jax import lax
from jax.experimental import pallas as pl
from jax.experimental.pallas import tpu as pltpu
from jax.experimental.pallas import tpu_sc as plsc

VOCAB = 1000
DIM = 64
BATCH = 16384
SEQ = 200

VPAD = 1024
NC, NS, L = 2, 16, 16
NW = NC * NS
NH = 2
HBATCH = BATCH // NH
ROWS_PER_W = HBATCH // NW
CHUNK = 32
NCHUNK = ROWS_PER_W // CHUNK
FULL_VECS = SEQ // L
TAIL = SEQ - FULL_VECS * L
NCB = VPAD // 128
BT = 256
NBLK_H = HBATCH // BT
CPB = BT // CHUNK

_mesh = plsc.VectorSubcoreMesh(core_axis_name="c", subcore_axis_name="s")


def _make_sc_counts(base_row):
    @functools.partial(
        pl.kernel,
        mesh=_mesh,
        out_type=jax.ShapeDtypeStruct((NBLK_H, NCB, CPB, CHUNK, 128), jnp.float32),
        scratch_types=[
            pltpu.VMEM((CHUNK, SEQ), jnp.int32),
            pltpu.VMEM((CHUNK, SEQ), jnp.int32),
            pltpu.VMEM((CHUNK, SEQ), jnp.int32),
            pltpu.VMEM((CHUNK, SEQ), jnp.int32),
            pltpu.VMEM((NCB, CHUNK, 128), jnp.float32),
            pltpu.VMEM((NCB, CHUNK, 128), jnp.float32),
            pltpu.SemaphoreType.DMA,
            pltpu.SemaphoreType.DMA,
            pltpu.SemaphoreType.DMA,
            pltpu.SemaphoreType.DMA,
            pltpu.SemaphoreType.DMA,
            pltpu.SemaphoreType.DMA,
        ],
        compiler_params=pltpu.CompilerParams(needs_layout_passes=False),
    )
    def _sc_counts(
        tok_hbm, counts_hbm,
        tok_0, tok_1, tok_2, tok_3, cnt_a, cnt_b,
        ts_0, ts_1, ts_2, ts_3, sem_a, sem_b,
    ):
        wid = lax.axis_index("s") * NC + lax.axis_index("c")
        row0 = wid * ROWS_PER_W

        zerosf = jnp.zeros((L,), jnp.float32)
        iota16 = lax.iota(jnp.int32, L)
        tail_mask = iota16 >= (L - TAIL)
        plus1 = jnp.full((L,), 1.0, jnp.float32)
        minus1 = jnp.full((L,), -1.0, jnp.float32)

        def zero_buf(cnt_v):
            @plsc.parallel_loop(0, NCB * CHUNK, unroll=2)
            def _(i):
                cb = i // CHUNK
                r = i % CHUNK
                for j in range(128 // L):
                    cnt_v[cb, r, pl.ds(j * L, L)] = zerosf

        def scatter_pass(tok_v, cnt_v, val_vec):
            @plsc.parallel_loop(0, CHUNK, unroll=2)
            def _(r):
                rows = jnp.full((L,), r, jnp.int32)
                for j in range(FULL_VECS + 1):
                    if j < FULL_VECS:
                        tok = tok_v[r, pl.ds(j * L, L)]
                        mask = None
                    else:
                        tok = tok_v[r, pl.ds(SEQ - L, L)]
                        mask = tail_mask
                    cbv = lax.shift_right_logical(tok, 7)
                    lo = tok & jnp.int32(0x7F)
                    if mask is None:
                        plsc.addupdate_scatter(cnt_v, [cbv, rows, lo], val_vec)
                    else:
                        plsc.addupdate_scatter(
                            cnt_v, [cbv, rows, lo], val_vec, mask=mask
                        )

        toks = (tok_0, tok_1, tok_2, tok_3)
        tsems = (ts_0, ts_1, ts_2, ts_3)
        cnts = (cnt_a, cnt_b)
        csems = (sem_a, sem_b)

        def tok_src(c):
            start = base_row + row0 + c * CHUNK
            return tok_hbm.at[pl.ds(start, CHUNK), :]

        def start_tok(c, i):
            pltpu.make_async_copy(tok_src(c), toks[i], tsems[i]).start()

        def wait_tok(i):
            pltpu.make_async_copy(tok_src(0), toks[i], tsems[i]).wait()

        def cnt_dst(c):
            crow = row0 + c * CHUNK
            return counts_hbm.at[crow // BT, :, (crow % BT) // CHUNK]

        def start_cnt_dma(c, cnt_v, sem):
            pltpu.make_async_copy(cnt_v, cnt_dst(c), sem).start()

        def wait_cnt_dma(cnt_v, sem):
            pltpu.make_async_copy(cnt_v, cnt_dst(0), sem).wait()

        for i in range(4):
            start_tok(i, i)
        zero_buf(cnt_a)
        zero_buf(cnt_b)
        for c in range(NCHUNK):
            tb = c % 4
            kb = c % 2
            if c >= 2:
                wait_cnt_dma(cnts[kb], csems[kb])
                scatter_pass(toks[(c - 2) % 4], cnts[kb], minus1)
                if c + 2 < NCHUNK:
                    start_tok(c + 2, (c + 2) % 4)
            wait_tok(tb)
            scatter_pass(toks[tb], cnts[kb], plus1)
            start_cnt_dma(c, cnts[kb], csems[kb])
        wait_cnt_dma(cnt_a, sem_a)
        wait_cnt_dma(cnt_b, sem_b)

    return _sc_counts


_sc_counts_halves = tuple(_make_sc_counts(h * HBATCH) for h in range(NH))


def _tc_matmul_body(counts_ref, emb_ref, out_ref):
    x = counts_ref[...]
    emb = emb_ref[...]
    acc = jnp.dot(x[0:BT, :], emb[0:128, :], preferred_element_type=jnp.float32)
    for cb in range(1, NCB):
        acc += jnp.dot(
            x[cb * BT : (cb + 1) * BT, :],
            emb[cb * 128 : (cb + 1) * 128, :],
            preferred_element_type=jnp.float32,
        )
    out_ref[...] = acc * (1.0 / SEQ)


def _tc_matmul(counts2d, emb_pad):
    return pl.pallas_call(
        _tc_matmul_body,
        grid=(NBLK_H,),
        in_specs=[
            pl.BlockSpec((NCB * BT, 128), lambda i: (i, 0)),
            pl.BlockSpec((VPAD, DIM), lambda i: (0, 0)),
        ],
        out_specs=pl.BlockSpec((BT, DIM), lambda i: (i, 0)),
        out_shape=jax.ShapeDtypeStruct((HBATCH, DIM), jnp.float32),
    )(counts2d, emb_pad)


def kernel(token_ids, emb_weight):
    emb_pad = jnp.pad(emb_weight, ((0, VPAD - VOCAB), (0, 0)))
    outs = []
    for h in range(NH):
        counts = _sc_counts_halves[h](token_ids)
        counts2d = counts.reshape(NBLK_H * NCB * BT, 128)
        outs.append(_tc_matmul(counts2d, emb_pad))
    return jnp.concatenate(outs, axis=0)


# speedup vs baseline: 1.1507x; 1.0088x over previous
import functools

import jax
import jax.numpy as jnp
from jax import lax
from jax.experimental import pallas as pl
from jax.experimental.pallas import tpu as pltpu
from jax.experimental.pallas import tpu_sc as plsc

VOCAB = 1000
DIM = 64
BATCH = 16384
SEQ = 200

VPAD = 1024
NC, NS, L = 2, 16, 16
NW = NC * NS
NH = 2
HBATCH = BATCH // NH
ROWS_PER_W = HBATCH // NW
CHUNK = 32
NCHUNK = ROWS_PER_W // CHUNK
FULL_VECS = SEQ // L
TAIL = SEQ - FULL_VECS * L
NCB = VPAD // 128
BT = 256
NBLK_H = HBATCH // BT
CPB = BT // CHUNK

_mesh = plsc.VectorSubcoreMesh(core_axis_name="c", subcore_axis_name="s")


def _make_sc_counts(base_row):
    @functools.partial(
        pl.kernel,
        mesh=_mesh,
        out_type=jax.ShapeDtypeStruct((NBLK_H, NCB, CPB, CHUNK, 128), jnp.float32),
        scratch_types=[
            pltpu.VMEM((CHUNK, SEQ), jnp.int32),
            pltpu.VMEM((CHUNK, SEQ), jnp.int32),
            pltpu.VMEM((CHUNK, SEQ), jnp.int32),
            pltpu.VMEM((CHUNK, SEQ), jnp.int32),
            pltpu.VMEM((NCB, CHUNK, 128), jnp.float32),
            pltpu.VMEM((NCB, CHUNK, 128), jnp.float32),
            pltpu.VMEM((NCB, CHUNK, 128), jnp.float32),
            pltpu.SemaphoreType.DMA,
            pltpu.SemaphoreType.DMA,
            pltpu.SemaphoreType.DMA,
            pltpu.SemaphoreType.DMA,
            pltpu.SemaphoreType.DMA,
            pltpu.SemaphoreType.DMA,
            pltpu.SemaphoreType.DMA,
        ],
        compiler_params=pltpu.CompilerParams(needs_layout_passes=False),
    )
    def _sc_counts(
        tok_hbm, counts_hbm,
        tok_0, tok_1, tok_2, tok_3, cnt_a, cnt_b, cnt_c,
        ts_0, ts_1, ts_2, ts_3, sem_a, sem_b, sem_c,
    ):
        wid = lax.axis_index("s") * NC + lax.axis_index("c")
        row0 = wid * ROWS_PER_W

        zerosf = jnp.zeros((L,), jnp.float32)
        iota16 = lax.iota(jnp.int32, L)
        tail_mask = iota16 >= (L - TAIL)
        plus1 = jnp.full((L,), 1.0, jnp.float32)
        minus1 = jnp.full((L,), -1.0, jnp.float32)

        def zero_buf(cnt_v):
            @plsc.parallel_loop(0, NCB * CHUNK, unroll=2)
            def _(i):
                cb = i // CHUNK
                r = i % CHUNK
                for j in range(128 // L):
                    cnt_v[cb, r, pl.ds(j * L, L)] = zerosf

        def scatter_pass(tok_v, cnt_v, val_vec):
            @plsc.parallel_loop(0, CHUNK, unroll=2)
            def _(r):
                rows = jnp.full((L,), r, jnp.int32)
                for j in range(FULL_VECS + 1):
                    if j < FULL_VECS:
                        tok = tok_v[r, pl.ds(j * L, L)]
                        mask = None
                    else:
                        tok = tok_v[r, pl.ds(SEQ - L, L)]
                        mask = tail_mask
                    cbv = lax.shift_right_logical(tok, 7)
                    lo = tok & jnp.int32(0x7F)
                    if mask is None:
                        plsc.addupdate_scatter(cnt_v, [cbv, rows, lo], val_vec)
                    else:
                        plsc.addupdate_scatter(
                            cnt_v, [cbv, rows, lo], val_vec, mask=mask
                        )

        toks = (tok_0, tok_1, tok_2, tok_3)
        tsems = (ts_0, ts_1, ts_2, ts_3)
        cnts = (cnt_a, cnt_b, cnt_c)
        csems = (sem_a, sem_b, sem_c)

        def tok_src(c):
            start = base_row + row0 + c * CHUNK
            return tok_hbm.at[pl.ds(start, CHUNK), :]

        def start_tok(c, i):
            pltpu.make_async_copy(tok_src(c), toks[i], tsems[i]).start()

        def wait_tok(i):
            pltpu.make_async_copy(tok_src(0), toks[i], tsems[i]).wait()

        def cnt_dst(c):
            crow = row0 + c * CHUNK
            return counts_hbm.at[crow // BT, :, (crow % BT) // CHUNK]

        def start_cnt_dma(c, cnt_v, sem):
            pltpu.make_async_copy(cnt_v, cnt_dst(c), sem).start()

        def wait_cnt_dma(cnt_v, sem):
            pltpu.make_async_copy(cnt_v, cnt_dst(0), sem).wait()

        for i in range(4):
            start_tok(i, i)
        zero_buf(cnt_a)
        zero_buf(cnt_b)
        zero_buf(cnt_c)
        for c in range(NCHUNK):
            tb = c % 4
            kb = c % 3
            if c >= 3:
                wait_cnt_dma(cnts[kb], csems[kb])
                scatter_pass(toks[(c - 3) % 4], cnts[kb], minus1)
                if c + 1 < NCHUNK:
                    start_tok(c + 1, (c + 1) % 4)
            wait_tok(tb)
            scatter_pass(toks[tb], cnts[kb], plus1)
            start_cnt_dma(c, cnts[kb], csems[kb])
        for kb in range(3):
            wait_cnt_dma(cnts[kb], csems[kb])

    return _sc_counts


_sc_counts_halves = tuple(_make_sc_counts(h * HBATCH) for h in range(NH))


def _tc_matmul_body(counts_ref, emb_ref, out_ref):
    x = counts_ref[...]
    emb = emb_ref[...]
    acc = jnp.dot(x[0:BT, :], emb[0:128, :], preferred_element_type=jnp.float32)
    for cb in range(1, NCB):
        acc += jnp.dot(
            x[cb * BT : (cb + 1) * BT, :],
            emb[cb * 128 : (cb + 1) * 128, :],
            preferred_element_type=jnp.float32,
        )
    out_ref[...] = acc * (1.0 / SEQ)


def _tc_matmul(counts2d, emb_pad):
    return pl.pallas_call(
        _tc_matmul_body,
        grid=(NBLK_H,),
        in_specs=[
            pl.BlockSpec((NCB * BT, 128), lambda i: (i, 0)),
            pl.BlockSpec((VPAD, DIM), lambda i: (0, 0)),
        ],
        out_specs=pl.BlockSpec((BT, DIM), lambda i: (i, 0)),
        out_shape=jax.ShapeDtypeStruct((HBATCH, DIM), jnp.float32),
    )(counts2d, emb_pad)


def kernel(token_ids, emb_weight):
    emb_pad = jnp.pad(emb_weight, ((0, VPAD - VOCAB), (0, 0)))
    outs = []
    for h in range(NH):
        counts = _sc_counts_halves[h](token_ids)
        counts2d = counts.reshape(NBLK_H * NCB * BT, 128)
        outs.append(_tc_matmul(counts2d, emb_pad))
    return jnp.concatenate(outs, axis=0)
